# baseline (speedup 1.0000x reference)
import functools
import math

import jax
import jax.numpy as jnp
from jax import lax
from jax.experimental import pallas as pl
from jax.experimental.pallas import tpu as pltpu
from jax.experimental.pallas import tpu_sc as plsc

_VOCAB = 1000000
_D = 64
_B = 4096
_S = 200

_NC, _NS = 2, 16
_NW = _NC * _NS
_ROWS = _B * _S
_N_PER_W = _ROWS // _NW
_CHUNK = 2 * _S
_SECS = _CHUNK // _S
_NCHUNK = _N_PER_W // _CHUNK


def _pe_table():
    position = jnp.arange(0.0, _S, dtype=jnp.float32)[:, None]
    div_term = jnp.exp(
        jnp.arange(0, _D, 2, dtype=jnp.float32) * (-(math.log(10000.0) / _D)))
    pe = jnp.zeros((_S, _D), dtype=jnp.float32)
    pe = pe.at[:, 0::2].set(jnp.sin(position * div_term))
    pe = pe.at[:, 1::2].set(jnp.cos(position * div_term))
    return pe


_mesh = plsc.VectorSubcoreMesh(
    core_axis_name="c", subcore_axis_name="s", num_cores=_NC, num_subcores=_NS)


@functools.partial(
    pl.kernel,
    out_type=jax.ShapeDtypeStruct((_ROWS, _D), jnp.float32),
    mesh=_mesh,
    scratch_types=[
        pltpu.VMEM((_CHUNK,), jnp.int32),
        pltpu.VMEM((_CHUNK, _D), jnp.float32),
        pltpu.VMEM((_S, _D), jnp.float32),
        pltpu.SemaphoreType.DMA,
    ],
    compiler_params=pltpu.CompilerParams(use_tc_tiling_on_sc=False),
)
def _embed(table_hbm, idx_hbm, pe_hbm, out_hbm, idx_v, rows_v, pe_v, sem):
    wid = lax.axis_index("s") * _NC + lax.axis_index("c")
    base = wid * _N_PER_W
    pltpu.sync_copy(pe_hbm, pe_v)

    def chunk_body(c, carry):
        rbase = base + c * _CHUNK
        pltpu.sync_copy(idx_hbm.at[pl.ds(rbase, _CHUNK)], idx_v)
        pltpu.async_copy(table_hbm.at[idx_v], rows_v, sem).wait()

        def add_body(pr, carry2):
            for col in range(_D // 16):
                pe_reg = pe_v[pr, pl.ds(col * 16, 16)]
                for s_ in range(_SECS):
                    plsc.addupdate(
                        rows_v.at[s_ * _S + pr, pl.ds(col * 16, 16)], pe_reg)
            return carry2

        lax.fori_loop(0, _S, add_body, 0)
        pltpu.sync_copy(rows_v, out_hbm.at[pl.ds(rbase, _CHUNK)])
        return carry

    lax.fori_loop(0, _NCHUNK, chunk_body, 0)


def kernel(indices, table):
    idx_flat = indices.reshape(_ROWS)
    out = _embed(table, idx_flat, _pe_table())
    return out.reshape(_B, _S, _D)
